# baseline (speedup 1.0000x reference)
import functools

import jax
import jax.numpy as jnp
from jax import lax
from jax.experimental import pallas as pl

_G = 3
_SUB = 4
_COLS = 16


def _compute_body(x_ref, y0_ref, y1_ref, y2_ref, tt_ref, *, k_groups):
    x = x_ref[...]
    n, d = x.shape
    hi_p = lax.Precision.HIGHEST
    ones_row = jnp.ones((1, d), jnp.float32)
    nx = lax.dot_general(ones_row, x * x, (((1,), (1,)), ((), ())),
                         precision=hi_p)
    parts = []
    for y_ref in (y0_ref, y1_ref, y2_ref):
        yp = y_ref[...]
        nyp = jnp.sum(yp * yp, axis=1, keepdims=True)
        g = lax.dot_general(yp, x, (((1,), (1,)), ((), ())),
                            precision=hi_p)
        parts.append(nx + nyp - 2.0 * g)
    r2 = jnp.concatenate(parts, axis=1)
    rows, w = r2.shape

    def triple_sum_aligned(a):
        s = a + jnp.roll(a, -1, 1) + jnp.roll(a, -2, 1)
        e = lax.broadcasted_iota(jnp.int32, (rows, w), 1) % _G
        return jnp.where(e == 0, s,
                         jnp.where(e == 1, jnp.roll(s, 1, 1),
                                   jnp.roll(s, 2, 1)))

    mean_exp = triple_sum_aligned(r2) * (1.0 / _G)
    dm = r2 - mean_exp
    var_exp = triple_sum_aligned(dm * dm) * (1.0 / _G)
    std_exp = jnp.sqrt(var_exp)
    out_exp = dm / std_exp

    q_i = lax.broadcasted_iota(jnp.int32, (w, n), 0)
    t_i = lax.broadcasted_iota(jnp.int32, (w, n), 1)

    def pe_mat(e):
        return (q_i == _G * t_i + e).astype(jnp.float32)

    p0 = pe_mat(0)
    std_g = lax.dot_general(std_exp, p0, (((1,), (0,)), ((), ())),
                            precision=hi_p)

    bits = lax.bitcast_convert_type(std_g, jnp.int32)

    def step(_, lh):
        lo, hi = lh
        mid = lo + (hi - lo + 1) // 2
        cnt = jnp.sum((bits >= mid).astype(jnp.int32))
        good = cnt >= k_groups
        return (jnp.where(good, mid, lo), jnp.where(good, hi, mid - 1))

    lo, _ = lax.fori_loop(0, 31, step,
                          (jnp.int32(0), jnp.int32(2**31 - 10)))
    mask_g = (bits >= lo).astype(jnp.float32)

    c0 = lax.dot_general(out_exp, p0, (((1,), (0,)), ((), ())))
    c1 = lax.dot_general(out_exp, pe_mat(1), (((1,), (0,)), ((), ())))
    c2 = lax.dot_general(out_exp, pe_mat(2), (((1,), (0,)), ((), ())))
    call = jnp.concatenate([c0, c1, c2, mask_g], axis=0)

    kk = call.shape[0]
    c_i = lax.broadcasted_iota(jnp.int32, (kk, kk), 0)
    k_i = lax.broadcasted_iota(jnp.int32, (kk, kk), 1)
    u_ = c_i // _COLS
    w_ = c_i % _COLS
    k_out = rows * (w_ % _G) + _SUB * u_ + w_ // _G
    k_msk = _G * rows + _SUB * u_ + (w_ - 12)
    e_all = (k_i == jnp.where(w_ < 12, k_out, k_msk)).astype(jnp.float32)
    ttp = lax.dot_general(e_all, call, (((1,), (0,)), ((), ())))
    tt_ref[...] = ttp.reshape(tt_ref.shape)


def _write_body(tt_ref, out_ref, mask_ref):
    eye16 = (lax.broadcasted_iota(jnp.int32, (_COLS, _COLS), 0)
             == lax.broadcasted_iota(jnp.int32, (_COLS, _COLS), 1)
             ).astype(jnp.float32)
    outs = []
    masks = []
    for half in range(4):
        slab = tt_ref[half]
        v16 = lax.dot_general(slab, eye16, (((0,), (0,)), ((), ())))
        for rho in range(_SUB):
            outs.append(v16[:, _G * rho:_G * rho + _G])
            masks.append(v16[:, 12 + rho:13 + rho])
    out_ref[...] = jnp.concatenate(outs, axis=0)
    mask_t = jnp.concatenate(masks, axis=0)
    mask_ref[...] = jnp.where(mask_t > 0.5, 1.0, 0.0)


def kernel(x, y):
    n, d = x.shape
    m = y.shape[0]
    rows = m // _G
    n_groups = (m * n) // _G
    k_groups = int(0.3 * n_groups)
    n_steps = n_groups // (_SUB * n)

    y3 = y.reshape(rows, _G, d)
    y0, y1, y2 = y3[:, 0], y3[:, 1], y3[:, 2]

    body = functools.partial(_compute_body, k_groups=k_groups)
    tt = pl.pallas_call(
        body,
        out_shape=jax.ShapeDtypeStruct((n_steps, _COLS, n), jnp.float32),
    )(x, y0, y1, y2)

    out, mask = pl.pallas_call(
        _write_body,
        grid=(n_steps // 4,),
        in_specs=[
            pl.BlockSpec((4, _COLS, n), lambda u: (u, 0, 0)),
        ],
        out_specs=[
            pl.BlockSpec((4 * _SUB * n, _G), lambda u: (u, 0)),
            pl.BlockSpec((4 * _SUB * n, 1), lambda u: (u, 0)),
        ],
        out_shape=[
            jax.ShapeDtypeStruct((n_groups, _G), jnp.float32),
            jax.ShapeDtypeStruct((n_groups, 1), jnp.float32),
        ],
    )(tt)
    return out, mask


# speedup vs baseline: 1.0118x; 1.0118x over previous
import functools

import jax
import jax.numpy as jnp
from jax import lax
from jax.experimental import pallas as pl

_G = 3
_SUB = 4
_COLS = 16


def _compute_body(x_ref, y0_ref, y1_ref, y2_ref, tt_ref, *, k_groups):
    x = x_ref[...]
    n, d = x.shape
    hi_p = lax.Precision.HIGHEST
    ones_row = jnp.ones((1, d), jnp.float32)
    nx = lax.dot_general(ones_row, x * x, (((1,), (1,)), ((), ())),
                         precision=hi_p)
    parts = []
    for y_ref in (y0_ref, y1_ref, y2_ref):
        yp = y_ref[...]
        nyp = jnp.sum(yp * yp, axis=1, keepdims=True)
        g = lax.dot_general(yp, x, (((1,), (1,)), ((), ())),
                            precision=hi_p)
        parts.append(nx + nyp - 2.0 * g)
    r2 = jnp.concatenate(parts, axis=1)
    rows, w = r2.shape

    def triple_sum_aligned(a):
        s = a + jnp.roll(a, -1, 1) + jnp.roll(a, -2, 1)
        e = lax.broadcasted_iota(jnp.int32, (rows, w), 1) % _G
        return jnp.where(e == 0, s,
                         jnp.where(e == 1, jnp.roll(s, 1, 1),
                                   jnp.roll(s, 2, 1)))

    mean_exp = triple_sum_aligned(r2) * (1.0 / _G)
    dm = r2 - mean_exp
    var_exp = triple_sum_aligned(dm * dm) * (1.0 / _G)
    std_exp = jnp.sqrt(var_exp)
    out_exp = dm / std_exp

    q_i = lax.broadcasted_iota(jnp.int32, (w, n), 0)
    t_i = lax.broadcasted_iota(jnp.int32, (w, n), 1)

    def pe_mat(e):
        return (q_i == _G * t_i + e).astype(jnp.float32)

    p0 = pe_mat(0)
    std_g = lax.dot_general(std_exp, p0, (((1,), (0,)), ((), ())),
                            precision=hi_p)

    bits = lax.bitcast_convert_type(std_g, jnp.int32)

    def step(_, lh):
        lo, hi = lh
        q = jnp.maximum((hi - lo + 1) // 4, 1)
        t3 = lo + q
        t2 = lo + 2 * q
        t1 = lo + 3 * q
        c1 = jnp.sum((bits >= t1).astype(jnp.int32))
        c2 = jnp.sum((bits >= t2).astype(jnp.int32))
        c3 = jnp.sum((bits >= t3).astype(jnp.int32))
        g1 = c1 >= k_groups
        g2 = c2 >= k_groups
        g3 = c3 >= k_groups
        new_lo = jnp.where(g1, t1, jnp.where(g2, t2, jnp.where(g3, t3, lo)))
        new_hi = jnp.where(g1, hi,
                           jnp.where(g2, t1 - 1,
                                     jnp.where(g3, t2 - 1, t3 - 1)))
        return (new_lo, new_hi)

    lo, _ = lax.fori_loop(0, 18, step,
                          (jnp.int32(0), jnp.int32(2**31 - 10)))
    mask_g = (bits >= lo).astype(jnp.float32)

    c0 = lax.dot_general(out_exp, p0, (((1,), (0,)), ((), ())))
    c1 = lax.dot_general(out_exp, pe_mat(1), (((1,), (0,)), ((), ())))
    c2 = lax.dot_general(out_exp, pe_mat(2), (((1,), (0,)), ((), ())))
    call = jnp.concatenate([c0, c1, c2, mask_g], axis=0)

    kk = call.shape[0]
    c_i = lax.broadcasted_iota(jnp.int32, (kk, kk), 0)
    k_i = lax.broadcasted_iota(jnp.int32, (kk, kk), 1)
    u_ = c_i // _COLS
    w_ = c_i % _COLS
    k_out = rows * (w_ % _G) + _SUB * u_ + w_ // _G
    k_msk = _G * rows + _SUB * u_ + (w_ - 12)
    e_all = (k_i == jnp.where(w_ < 12, k_out, k_msk)).astype(jnp.float32)
    ttp = lax.dot_general(e_all, call, (((1,), (0,)), ((), ())))
    tt_ref[...] = ttp.reshape(tt_ref.shape)


def _write_body(tt_ref, out_ref, mask_ref):
    eye16 = (lax.broadcasted_iota(jnp.int32, (_COLS, _COLS), 0)
             == lax.broadcasted_iota(jnp.int32, (_COLS, _COLS), 1)
             ).astype(jnp.float32)
    outs = []
    masks = []
    for half in range(4):
        slab = tt_ref[half]
        v16 = lax.dot_general(slab, eye16, (((0,), (0,)), ((), ())))
        for rho in range(_SUB):
            outs.append(v16[:, _G * rho:_G * rho + _G])
            masks.append(v16[:, 12 + rho:13 + rho])
    out_ref[...] = jnp.concatenate(outs, axis=0)
    mask_t = jnp.concatenate(masks, axis=0)
    mask_ref[...] = jnp.where(mask_t > 0.5, 1.0, 0.0)


def kernel(x, y):
    n, d = x.shape
    m = y.shape[0]
    rows = m // _G
    n_groups = (m * n) // _G
    k_groups = int(0.3 * n_groups)
    n_steps = n_groups // (_SUB * n)

    y3 = y.reshape(rows, _G, d)
    y0, y1, y2 = y3[:, 0], y3[:, 1], y3[:, 2]

    body = functools.partial(_compute_body, k_groups=k_groups)
    tt = pl.pallas_call(
        body,
        out_shape=jax.ShapeDtypeStruct((n_steps, _COLS, n), jnp.float32),
    )(x, y0, y1, y2)

    out, mask = pl.pallas_call(
        _write_body,
        grid=(n_steps // 4,),
        in_specs=[
            pl.BlockSpec((4, _COLS, n), lambda u: (u, 0, 0)),
        ],
        out_specs=[
            pl.BlockSpec((4 * _SUB * n, _G), lambda u: (u, 0)),
            pl.BlockSpec((4 * _SUB * n, 1), lambda u: (u, 0)),
        ],
        out_shape=[
            jax.ShapeDtypeStruct((n_groups, _G), jnp.float32),
            jax.ShapeDtypeStruct((n_groups, 1), jnp.float32),
        ],
    )(tt)
    return out, mask
